# baseline (speedup 1.0000x reference)
import jax
import jax.numpy as jnp
from jax import lax
from jax.experimental import pallas as pl
from jax.experimental.pallas import tpu as pltpu
from jax.experimental.pallas import tpu_sc as plsc

N = 10000
E = 320000
D_IN = 192
C_OUT = 2

NC = 2
NS = 16
L = 16
NW = NC * NS

BLK = 128
NB = E // BLK
NBT = NB // NW
NXT = NB - NBT * NW
GPB = BLK // L
NSL = ((N + NW - 1) // NW + L - 1) // L * L
NP2 = 10240
CHK = NP2 // NS
ZIT = NP2 // L


def _prologue_body(xt_ref, wt_ref, as_ref, ad_ref,
                   asv_ref, adv_ref, h0_ref, h1_ref, sh_ref):
    ht = lax.dot_general(wt_ref[...], xt_ref[...],
                         dimension_numbers=(((1,), (0,)), ((), ())),
                         preferred_element_type=jnp.float32)
    asv = ht[0:1] * as_ref[0, 0] + ht[1:2] * as_ref[0, 1]
    adv = ht[0:1] * ad_ref[0, 0] + ht[1:2] * ad_ref[0, 1]
    asv_ref[...] = asv[0]
    adv_ref[...] = adv[0]
    h0_ref[...] = ht[0]
    h1_ref[...] = ht[1]
    m = jnp.max(asv) + jnp.max(adv)
    sh = jnp.where(m >= 0.0, m, 0.2 * m)
    sh_ref[...] = jnp.full((L,), sh, jnp.float32)


def _sc_body(as_hbm, ad_hbm, h0_hbm, h1_hbm, sh_hbm, ei_hbm, i3_hbm,
             parts_hbm,
             as_v, ad_v, h0_v, h1_v, sh_v, ei_v,
             acc_v, i3_v, sh3, sem):
    cid = lax.axis_index("c")
    sid = lax.axis_index("s")
    wid = sid * NC + cid
    base_b = wid * NBT + jnp.minimum(wid, NXT)
    base = pl.multiple_of(base_b * (2 * BLK), 8)
    has_extra = wid < NXT
    copies = [
        pltpu.async_copy(as_hbm, as_v, sem),
        pltpu.async_copy(ad_hbm, ad_v, sem),
        pltpu.async_copy(h0_hbm, h0_v, sem),
        pltpu.async_copy(h1_hbm, h1_v, sem),
        pltpu.async_copy(sh_hbm, sh_v, sem),
        pltpu.async_copy(i3_hbm, i3_v, sem),
        pltpu.async_copy(ei_hbm.at[pl.ds(base, NBT * 2 * BLK)],
                         ei_v.at[pl.ds(0, NBT * 2 * BLK)], sem),
    ]

    @pl.when(has_extra)
    def _():
        pltpu.sync_copy(ei_hbm.at[pl.ds(base + NBT * 2 * BLK, 2 * BLK)],
                        ei_v.at[pl.ds(NBT * 2 * BLK, 2 * BLK)])

    z = jnp.zeros((L,), jnp.float32)

    def zero_step(i, carry):
        off = pl.multiple_of(i * L, 8)
        acc_v[0, pl.ds(off, L)] = z
        acc_v[1, pl.ds(off, L)] = z
        acc_v[2, pl.ds(off, L)] = z
        return carry

    lax.fori_loop(0, ZIT, zero_step, 0)

    @pl.when(sid == 0)
    def _():
        pltpu.sync_copy(acc_v, sh3)

    for c in copies:
        c.wait()

    shift = sh_v[...]
    r0 = jnp.zeros((L,), jnp.int32)
    r1 = r0 + 1
    r2 = r0 + 2

    def do_group(off_s):
        s = ei_v[pl.ds(off_s, L)]
        d = ei_v[pl.ds(off_s + BLK, L)]
        av = plsc.load_gather(as_v, [s]) + plsc.load_gather(ad_v, [d])
        av = jnp.maximum(av, 0.2 * av) - shift
        ex = jnp.exp(av)
        h0 = plsc.load_gather(h0_v, [s])
        h1 = plsc.load_gather(h1_v, [s])
        plsc.addupdate_scatter(acc_v, [r0, d], ex)
        plsc.addupdate_scatter(acc_v, [r1, d], ex * h0)
        plsc.addupdate_scatter(acc_v, [r2, d], ex * h1)

    def step(i, carry):
        block = pl.multiple_of(i * (2 * BLK), 8)
        for g in range(GPB):
            do_group(block + g * L)
        return carry

    lax.fori_loop(0, NBT, step, 0)

    @pl.when(has_extra)
    def _():
        for g in range(GPB):
            do_group(NBT * 2 * BLK + g * L)

    nbase = pl.multiple_of(wid * NSL, 8)
    nits = (jnp.minimum(NSL, N - nbase) + L - 1) // L

    def loop_step(j, carry):
        sl = pl.ds(nbase + j * L, L)
        av = as_v[sl] + ad_v[sl]
        av = jnp.maximum(av, 0.2 * av) - shift
        ex = jnp.exp(av)
        plsc.addupdate(acc_v.at[0, sl], ex)
        plsc.addupdate(acc_v.at[1, sl], ex * h0_v[sl])
        plsc.addupdate(acc_v.at[2, sl], ex * h1_v[sl])
        return carry

    lax.fori_loop(0, nits, loop_step, 0)

    plsc.subcore_barrier()
    pltpu.sync_copy(acc_v, sh3.at[i3_v], add=True)
    plsc.subcore_barrier()
    nb = pl.multiple_of(sid * CHK, 8)
    pltpu.sync_copy(sh3.at[:, pl.ds(nb, CHK)],
                    parts_hbm.at[cid, :, pl.ds(nb, CHK)])


def _epilogue_body(p_ref, b_ref, out_ref):
    q = p_ref[0] + p_ref[1]
    den = q[0:1, :N]
    n0 = q[1:2, :N]
    n1 = q[2:3, :N]
    inv = 1.0 / (den + 1e-16)
    out_ref[...] = jnp.concatenate(
        [n0 * inv + b_ref[0, 0], n1 * inv + b_ref[0, 1]], axis=0)


@jax.jit
def kernel(x, edge_index, edge_attr, W, att_src, att_dst, bias):
    del edge_attr
    f32 = jnp.float32

    prologue = pl.pallas_call(
        _prologue_body,
        in_specs=[
            pl.BlockSpec((D_IN, N), lambda: (0, 0)),
            pl.BlockSpec((C_OUT, D_IN), lambda: (0, 0)),
            pl.BlockSpec(memory_space=pltpu.SMEM),
            pl.BlockSpec(memory_space=pltpu.SMEM),
        ],
        out_shape=[
            jax.ShapeDtypeStruct((N,), f32),
            jax.ShapeDtypeStruct((N,), f32),
            jax.ShapeDtypeStruct((N,), f32),
            jax.ShapeDtypeStruct((N,), f32),
            jax.ShapeDtypeStruct((L,), f32),
        ],
    )
    asv, adv, h0, h1, sh = prologue(
        x.T, W.T, att_src.reshape(1, C_OUT), att_dst.reshape(1, C_OUT))

    ei_flat = edge_index.reshape(2, NB, BLK).swapaxes(0, 1).reshape(2 * E)

    i3 = jnp.arange(3, dtype=jnp.int32)

    mesh = plsc.VectorSubcoreMesh(
        core_axis_name="c", subcore_axis_name="s", num_cores=NC, num_subcores=NS)
    sc = pl.kernel(
        _sc_body,
        out_type=jax.ShapeDtypeStruct((NC, 3, NP2), f32),
        mesh=mesh,
        compiler_params=pltpu.CompilerParams(
            needs_layout_passes=False, use_tc_tiling_on_sc=False),
        scratch_types=[
            pltpu.VMEM((N,), f32),
            pltpu.VMEM((N,), f32),
            pltpu.VMEM((N,), f32),
            pltpu.VMEM((N,), f32),
            pltpu.VMEM((L,), f32),
            pltpu.VMEM(((NBT + 1) * 2 * BLK,), jnp.int32),
            pltpu.VMEM((3, NP2), f32),
            pltpu.VMEM((3,), jnp.int32),
            pltpu.VMEM_SHARED((3, NP2), f32),
            pltpu.SemaphoreType.DMA,
        ],
    )
    parts = sc(asv, adv, h0, h1, sh, ei_flat, i3)

    epilogue = pl.pallas_call(
        _epilogue_body,
        in_specs=[
            pl.BlockSpec((NC, 3, NP2), lambda: (0, 0, 0)),
            pl.BlockSpec(memory_space=pltpu.SMEM),
        ],
        out_shape=jax.ShapeDtypeStruct((2, N), f32),
    )
    out2 = epilogue(parts, bias.reshape(1, C_OUT))
    return out2.T


# speedup vs baseline: 1.2242x; 1.2242x over previous
import jax
import jax.numpy as jnp
from jax import lax
from jax.experimental import pallas as pl
from jax.experimental.pallas import tpu as pltpu
from jax.experimental.pallas import tpu_sc as plsc

N = 10000
E = 320000
D_IN = 192
C_OUT = 2

NC = 2
NS = 16
L = 16
NW = NC * NS

U = 5
EPW = E // NW
NIT = EPW // (U * L)
NSL = ((N + NW - 1) // NW + L - 1) // L * L
NP2 = 10240
CHK = NP2 // NS
ZIT = NP2 // L


def _prologue_body(xt_ref, wt_ref, ei_ref, as_ref, ad_ref,
                   asv_ref, adv_ref, h0_ref, h1_ref, sh_ref,
                   src_ref, dst_ref):
    ht = lax.dot_general(wt_ref[...], xt_ref[...],
                         dimension_numbers=(((1,), (0,)), ((), ())),
                         preferred_element_type=jnp.float32)
    asv = ht[0:1] * as_ref[0, 0] + ht[1:2] * as_ref[0, 1]
    adv = ht[0:1] * ad_ref[0, 0] + ht[1:2] * ad_ref[0, 1]
    asv_ref[...] = asv[0]
    adv_ref[...] = adv[0]
    h0_ref[...] = ht[0]
    h1_ref[...] = ht[1]
    m = jnp.max(asv) + jnp.max(adv)
    sh = jnp.where(m >= 0.0, m, 0.2 * m)
    sh_ref[...] = jnp.full((L,), sh, jnp.float32)
    src_ref[...] = ei_ref[0]
    dst_ref[...] = ei_ref[1]


def _sc_body(as_hbm, ad_hbm, h0_hbm, h1_hbm, sh_hbm, src_hbm, dst_hbm, i3_hbm,
             parts_hbm,
             as_v, ad_v, h0_v, h1_v, sh_v, src_v, dst_v,
             acc_v, i3_v, sh3, sem):
    cid = lax.axis_index("c")
    sid = lax.axis_index("s")
    wid = sid * NC + cid
    base = pl.multiple_of(wid * EPW, 8)
    with jax.named_scope("sc_stage"):
        copies = [
            pltpu.async_copy(as_hbm, as_v, sem),
            pltpu.async_copy(ad_hbm, ad_v, sem),
            pltpu.async_copy(h0_hbm, h0_v, sem),
            pltpu.async_copy(h1_hbm, h1_v, sem),
            pltpu.async_copy(sh_hbm, sh_v, sem),
            pltpu.async_copy(i3_hbm, i3_v, sem),
            pltpu.async_copy(src_hbm.at[pl.ds(base, EPW)], src_v, sem),
            pltpu.async_copy(dst_hbm.at[pl.ds(base, EPW)], dst_v, sem),
        ]

        z = jnp.zeros((L,), jnp.float32)

        def zero_step(i, carry):
            off = pl.multiple_of(i * L, 8)
            acc_v[0, pl.ds(off, L)] = z
            acc_v[1, pl.ds(off, L)] = z
            acc_v[2, pl.ds(off, L)] = z
            return carry

        lax.fori_loop(0, ZIT, zero_step, 0)

        @pl.when(sid == 0)
        def _():
            pltpu.sync_copy(acc_v, sh3)

        for c in copies:
            c.wait()

    shift = sh_v[...]
    r0 = jnp.zeros((L,), jnp.int32)
    r1 = r0 + 1
    r2 = r0 + 2

    with jax.named_scope("sc_edges"):
        def step(i, carry):
            block = pl.multiple_of(i * (U * L), 8)
            for u in range(U):
                off = block + u * L
                s = src_v[pl.ds(off, L)]
                d = dst_v[pl.ds(off, L)]
                av = plsc.load_gather(as_v, [s]) + plsc.load_gather(ad_v, [d])
                av = jnp.maximum(av, 0.2 * av) - shift
                ex = jnp.exp(av)
                h0 = plsc.load_gather(h0_v, [s])
                h1 = plsc.load_gather(h1_v, [s])
                plsc.addupdate_scatter(acc_v, [r0, d], ex)
                plsc.addupdate_scatter(acc_v, [r1, d], ex * h0)
                plsc.addupdate_scatter(acc_v, [r2, d], ex * h1)
            return carry

        lax.fori_loop(0, NIT, step, 0)

    with jax.named_scope("sc_selfloop"):
        nbase = pl.multiple_of(wid * NSL, 8)
        nits = (jnp.minimum(NSL, N - nbase) + L - 1) // L

        def loop_step(j, carry):
            sl = pl.ds(nbase + j * L, L)
            av = as_v[sl] + ad_v[sl]
            av = jnp.maximum(av, 0.2 * av) - shift
            ex = jnp.exp(av)
            plsc.addupdate(acc_v.at[0, sl], ex)
            plsc.addupdate(acc_v.at[1, sl], ex * h0_v[sl])
            plsc.addupdate(acc_v.at[2, sl], ex * h1_v[sl])
            return carry

        lax.fori_loop(0, nits, loop_step, 0)

    with jax.named_scope("sc_combine"):
        plsc.subcore_barrier()
        pltpu.sync_copy(acc_v, sh3.at[i3_v], add=True)
        plsc.subcore_barrier()
    with jax.named_scope("sc_writeback"):
        nb = pl.multiple_of(sid * CHK, 8)
        pltpu.sync_copy(sh3.at[:, pl.ds(nb, CHK)],
                        parts_hbm.at[cid, :, pl.ds(nb, CHK)])


def _epilogue_body(p_ref, b_ref, out_ref):
    q = p_ref[0] + p_ref[1]
    den = q[0:1, :N]
    n0 = q[1:2, :N]
    n1 = q[2:3, :N]
    inv = 1.0 / (den + 1e-16)
    out_ref[...] = jnp.concatenate(
        [n0 * inv + b_ref[0, 0], n1 * inv + b_ref[0, 1]], axis=0)


@jax.jit
def kernel(x, edge_index, edge_attr, W, att_src, att_dst, bias):
    del edge_attr
    f32 = jnp.float32

    prologue = pl.pallas_call(
        _prologue_body,
        in_specs=[
            pl.BlockSpec((D_IN, N), lambda: (0, 0)),
            pl.BlockSpec((C_OUT, D_IN), lambda: (0, 0)),
            pl.BlockSpec((2, E), lambda: (0, 0)),
            pl.BlockSpec(memory_space=pltpu.SMEM),
            pl.BlockSpec(memory_space=pltpu.SMEM),
        ],
        out_shape=[
            jax.ShapeDtypeStruct((N,), f32),
            jax.ShapeDtypeStruct((N,), f32),
            jax.ShapeDtypeStruct((N,), f32),
            jax.ShapeDtypeStruct((N,), f32),
            jax.ShapeDtypeStruct((L,), f32),
            jax.ShapeDtypeStruct((E,), jnp.int32),
            jax.ShapeDtypeStruct((E,), jnp.int32),
        ],
    )
    asv, adv, h0, h1, sh, src_lin, dst_lin = prologue(
        x.T, W.T, edge_index,
        att_src.reshape(1, C_OUT), att_dst.reshape(1, C_OUT))

    i3 = jnp.arange(3, dtype=jnp.int32)

    mesh = plsc.VectorSubcoreMesh(
        core_axis_name="c", subcore_axis_name="s", num_cores=NC, num_subcores=NS)
    sc = pl.kernel(
        _sc_body,
        out_type=jax.ShapeDtypeStruct((NC, 3, NP2), f32),
        mesh=mesh,
        compiler_params=pltpu.CompilerParams(
            needs_layout_passes=False, use_tc_tiling_on_sc=False),
        scratch_types=[
            pltpu.VMEM((N,), f32),
            pltpu.VMEM((N,), f32),
            pltpu.VMEM((N,), f32),
            pltpu.VMEM((N,), f32),
            pltpu.VMEM((L,), f32),
            pltpu.VMEM((EPW,), jnp.int32),
            pltpu.VMEM((EPW,), jnp.int32),
            pltpu.VMEM((3, NP2), f32),
            pltpu.VMEM((3,), jnp.int32),
            pltpu.VMEM_SHARED((3, NP2), f32),
            pltpu.SemaphoreType.DMA,
        ],
    )
    parts = sc(asv, adv, h0, h1, sh, src_lin, dst_lin, i3)

    epilogue = pl.pallas_call(
        _epilogue_body,
        in_specs=[
            pl.BlockSpec((NC, 3, NP2), lambda: (0, 0, 0)),
            pl.BlockSpec(memory_space=pltpu.SMEM),
        ],
        out_shape=jax.ShapeDtypeStruct((2, N), f32),
    )
    out2 = epilogue(parts, bias.reshape(1, C_OUT))
    return out2.T


# speedup vs baseline: 1.5449x; 1.2619x over previous
import jax
import jax.numpy as jnp
import numpy as np
from jax import lax
from jax.experimental import pallas as pl
from jax.experimental.pallas import tpu as pltpu
from jax.experimental.pallas import tpu_sc as plsc

N = 10000
E = 320000
D_IN = 192
C_OUT = 2

NC = 2
NS = 16
L = 16
NW = NC * NS

U = 5
EPW = E // NW
NIT = EPW // (U * L)
NSL = ((N + NW - 1) // NW + L - 1) // L * L
NP2 = 10240
CHK = NP2 // NS
ZIT = NP2 // L


def _prologue_body(xt_ref, wt_ref, ei_ref, as_ref, ad_ref,
                   asv_ref, adv_ref, h0_ref, h1_ref, sh_ref,
                   src_ref, dst_ref):
    ht = lax.dot_general(wt_ref[...], xt_ref[...],
                         dimension_numbers=(((1,), (0,)), ((), ())),
                         preferred_element_type=jnp.float32)
    asv = ht[0:1] * as_ref[0, 0] + ht[1:2] * as_ref[0, 1]
    adv = ht[0:1] * ad_ref[0, 0] + ht[1:2] * ad_ref[0, 1]
    asv_ref[...] = asv[0]
    adv_ref[...] = adv[0]
    h0_ref[...] = ht[0]
    h1_ref[...] = ht[1]
    m = jnp.max(asv) + jnp.max(adv)
    sh = jnp.where(m >= 0.0, m, 0.2 * m)
    sh_ref[...] = jnp.full((L,), sh, jnp.float32)
    src_ref[...] = ei_ref[0]
    dst_ref[...] = ei_ref[1]


def _sc_body(as_hbm, ad_hbm, h0_hbm, h1_hbm, sh_hbm, src_hbm, dst_hbm, i3_hbm,
             parts_hbm,
             as_v, ad_v, h0_v, h1_v, sh_v, src_v, dst_v,
             acc_v, i3_v, sh3, tbl_sh, sem):
    cid = lax.axis_index("c")
    sid = lax.axis_index("s")
    wid = sid * NC + cid
    base = pl.multiple_of(wid * EPW, 8)
    with jax.named_scope("sc_stage"):
        copies = [
            pltpu.async_copy(sh_hbm, sh_v, sem),
            pltpu.async_copy(i3_hbm, i3_v, sem),
            pltpu.async_copy(src_hbm.at[pl.ds(base, EPW)], src_v, sem),
            pltpu.async_copy(dst_hbm.at[pl.ds(base, EPW)], dst_v, sem),
        ]

        @pl.when(sid == 0)
        def _():
            pltpu.sync_copy(as_hbm, tbl_sh.at[0])
        @pl.when(sid == 1)
        def _():
            pltpu.sync_copy(ad_hbm, tbl_sh.at[1])
        @pl.when(sid == 2)
        def _():
            pltpu.sync_copy(h0_hbm, tbl_sh.at[2])
        @pl.when(sid == 3)
        def _():
            pltpu.sync_copy(h1_hbm, tbl_sh.at[3])

        z = jnp.zeros((L,), jnp.float32)

        def zero_step(i, carry):
            off = pl.multiple_of(i * L, 8)
            acc_v[0, pl.ds(off, L)] = z
            acc_v[1, pl.ds(off, L)] = z
            acc_v[2, pl.ds(off, L)] = z
            return carry

        lax.fori_loop(0, ZIT, zero_step, 0)

        @pl.when(sid == 0)
        def _():
            pltpu.sync_copy(acc_v, sh3)

        plsc.subcore_barrier()
        pltpu.sync_copy(tbl_sh.at[0], as_v)
        pltpu.sync_copy(tbl_sh.at[1], ad_v)
        pltpu.sync_copy(tbl_sh.at[2], h0_v)
        pltpu.sync_copy(tbl_sh.at[3], h1_v)
        for c in copies:
            c.wait()

    shift = sh_v[...]
    r0 = jnp.zeros((L,), jnp.int32)
    r1 = r0 + 1
    r2 = r0 + 2

    with jax.named_scope("sc_edges"):
        @plsc.parallel_loop(0, EPW // L, 1, unroll=U)
        def _edges(i):
            off = pl.multiple_of(i * L, 8)
            s = src_v[pl.ds(off, L)]
            d = dst_v[pl.ds(off, L)]
            av = plsc.load_gather(as_v, [s]) + plsc.load_gather(ad_v, [d])
            av = jnp.maximum(av, 0.2 * av) - shift
            ex = jnp.exp(av)
            h0 = plsc.load_gather(h0_v, [s])
            h1 = plsc.load_gather(h1_v, [s])
            plsc.addupdate_scatter(acc_v, [r0, d], ex)
            plsc.addupdate_scatter(acc_v, [r1, d], ex * h0)
            plsc.addupdate_scatter(acc_v, [r2, d], ex * h1)

    with jax.named_scope("sc_selfloop"):
        nbase = pl.multiple_of(wid * NSL, 8)
        nits = (jnp.minimum(NSL, N - nbase) + L - 1) // L

        def loop_step(j, carry):
            sl = pl.ds(nbase + j * L, L)
            av = as_v[sl] + ad_v[sl]
            av = jnp.maximum(av, 0.2 * av) - shift
            ex = jnp.exp(av)
            plsc.addupdate(acc_v.at[0, sl], ex)
            plsc.addupdate(acc_v.at[1, sl], ex * h0_v[sl])
            plsc.addupdate(acc_v.at[2, sl], ex * h1_v[sl])
            return carry

        lax.fori_loop(0, nits, loop_step, 0)

    with jax.named_scope("sc_combine"):
        plsc.subcore_barrier()
        pltpu.sync_copy(acc_v, sh3.at[i3_v], add=True)
        plsc.subcore_barrier()
    with jax.named_scope("sc_writeback"):
        nb = pl.multiple_of(sid * CHK, 8)
        pltpu.sync_copy(sh3.at[:, pl.ds(nb, CHK)],
                        parts_hbm.at[cid, :, pl.ds(nb, CHK)])


def _epilogue_body(p_hbm, b_ref, out_ref, p_ref):
    pltpu.sync_copy(p_hbm, p_ref)
    q = p_ref[0] + p_ref[1]
    den = q[0:1, :N]
    n0 = q[1:2, :N]
    n1 = q[2:3, :N]
    inv = 1.0 / (den + 1e-16)
    out_ref[...] = jnp.concatenate(
        [n0 * inv + b_ref[0, 0], n1 * inv + b_ref[0, 1]], axis=0)


@jax.jit
def kernel(x, edge_index, edge_attr, W, att_src, att_dst, bias):
    del edge_attr
    f32 = jnp.float32

    prologue = pl.pallas_call(
        _prologue_body,
        in_specs=[
            pl.BlockSpec((D_IN, N), lambda: (0, 0)),
            pl.BlockSpec((C_OUT, D_IN), lambda: (0, 0)),
            pl.BlockSpec((2, E), lambda: (0, 0)),
            pl.BlockSpec(memory_space=pltpu.SMEM),
            pl.BlockSpec(memory_space=pltpu.SMEM),
        ],
        out_shape=[
            jax.ShapeDtypeStruct((N,), f32),
            jax.ShapeDtypeStruct((N,), f32),
            jax.ShapeDtypeStruct((N,), f32),
            jax.ShapeDtypeStruct((N,), f32),
            jax.ShapeDtypeStruct((L,), f32),
            jax.ShapeDtypeStruct((E,), jnp.int32),
            jax.ShapeDtypeStruct((E,), jnp.int32),
        ],
    )
    asv, adv, h0, h1, sh, src_lin, dst_lin = prologue(
        x.T, W.T, edge_index,
        att_src.reshape(1, C_OUT), att_dst.reshape(1, C_OUT))

    i3 = jnp.asarray(np.arange(3, dtype=np.int32))

    mesh = plsc.VectorSubcoreMesh(
        core_axis_name="c", subcore_axis_name="s", num_cores=NC, num_subcores=NS)
    sc = pl.kernel(
        _sc_body,
        out_type=jax.ShapeDtypeStruct((NC, 3, NP2), f32),
        mesh=mesh,
        compiler_params=pltpu.CompilerParams(
            needs_layout_passes=False, use_tc_tiling_on_sc=False),
        scratch_types=[
            pltpu.VMEM((N,), f32),
            pltpu.VMEM((N,), f32),
            pltpu.VMEM((N,), f32),
            pltpu.VMEM((N,), f32),
            pltpu.VMEM((L,), f32),
            pltpu.VMEM((EPW,), jnp.int32),
            pltpu.VMEM((EPW,), jnp.int32),
            pltpu.VMEM((3, NP2), f32),
            pltpu.VMEM((3,), jnp.int32),
            pltpu.VMEM_SHARED((3, NP2), f32),
            pltpu.VMEM_SHARED((4, N), f32),
            pltpu.SemaphoreType.DMA,
        ],
    )
    parts = sc(asv, adv, h0, h1, sh, src_lin, dst_lin, i3)

    epilogue = pl.pallas_call(
        _epilogue_body,
        in_specs=[
            pl.BlockSpec(memory_space=pl.ANY),
            pl.BlockSpec(memory_space=pltpu.SMEM),
        ],
        scratch_shapes=[pltpu.VMEM((NC, 3, NP2), f32)],
        out_shape=jax.ShapeDtypeStruct((2, N), f32),
    )
    out2 = epilogue(parts, bias.reshape(1, C_OUT))
    return out2.T


# speedup vs baseline: 1.7159x; 1.1107x over previous
import jax
import jax.numpy as jnp
import numpy as np
from jax import lax
from jax.experimental import pallas as pl
from jax.experimental.pallas import tpu as pltpu
from jax.experimental.pallas import tpu_sc as plsc

N = 10000
E = 320000
D_IN = 192
C_OUT = 2

NC = 2
NS = 16
L = 16
NW = NC * NS

U = 5
EPW = E // NW
NIT = EPW // (U * L)
NSL = ((N + NW - 1) // NW + L - 1) // L * L
NP2 = 10240
CHK = NP2 // NS
ZIT = NP2 // L


def _prologue_body(xt_ref, wt_ref, ei_ref, as_ref, ad_ref,
                   asv_ref, adv_ref, h0_ref, h1_ref, sh_ref,
                   src_ref, dst_ref):
    ht = lax.dot_general(wt_ref[...], xt_ref[...],
                         dimension_numbers=(((1,), (0,)), ((), ())),
                         preferred_element_type=jnp.float32)
    asv = ht[0:1] * as_ref[0, 0] + ht[1:2] * as_ref[0, 1]
    adv = ht[0:1] * ad_ref[0, 0] + ht[1:2] * ad_ref[0, 1]
    asv_ref[...] = asv[0]
    adv_ref[...] = adv[0]
    h0_ref[...] = ht[0]
    h1_ref[...] = ht[1]
    m = jnp.max(asv) + jnp.max(adv)
    sh = jnp.where(m >= 0.0, m, 0.2 * m)
    sh_ref[...] = jnp.full((L,), sh, jnp.float32)
    src_ref[...] = ei_ref[0]
    dst_ref[...] = ei_ref[1]


def _sc_body(as_hbm, ad_hbm, h0_hbm, h1_hbm, sh_hbm, src_hbm, dst_hbm, i3_hbm,
             z3_hbm, parts_hbm,
             tbl_v, sh_v, src_v, dst_v,
             acc_v, i3_v, sh3, tbl_sh, sem):
    cid = lax.axis_index("c")
    sid = lax.axis_index("s")
    wid = sid * NC + cid
    base = pl.multiple_of(wid * EPW, 8)
    with jax.named_scope("sc_stage"):
        copies = [
            pltpu.async_copy(sh_hbm, sh_v, sem),
            pltpu.async_copy(i3_hbm, i3_v, sem),
            pltpu.async_copy(src_hbm.at[pl.ds(base, EPW)], src_v, sem),
            pltpu.async_copy(dst_hbm.at[pl.ds(base, EPW)], dst_v, sem),
        ]

        @pl.when(sid == 0)
        def _():
            pltpu.sync_copy(z3_hbm, sh3)
        @pl.when(sid == 1)
        def _():
            pltpu.sync_copy(as_hbm, tbl_sh.at[0])
        @pl.when(sid == 2)
        def _():
            pltpu.sync_copy(ad_hbm, tbl_sh.at[1])
        @pl.when(sid == 3)
        def _():
            pltpu.sync_copy(h0_hbm, tbl_sh.at[2])
        @pl.when(sid == 4)
        def _():
            pltpu.sync_copy(h1_hbm, tbl_sh.at[3])

        z = jnp.zeros((L,), jnp.float32)

        @plsc.parallel_loop(0, ZIT, 1, unroll=4)
        def _zero(i):
            off = pl.multiple_of(i * L, 8)
            acc_v[0, pl.ds(off, L)] = z
            acc_v[1, pl.ds(off, L)] = z
            acc_v[2, pl.ds(off, L)] = z

        plsc.subcore_barrier()
        pltpu.sync_copy(tbl_sh, tbl_v)
        for c in copies:
            c.wait()

    shift = sh_v[...]
    r0 = jnp.zeros((L,), jnp.int32)
    r1 = r0 + 1
    r2 = r0 + 2
    r3 = r0 + 3

    with jax.named_scope("sc_edges"):
        @plsc.parallel_loop(0, EPW // L, 1, unroll=U)
        def _edges(i):
            off = pl.multiple_of(i * L, 8)
            s = src_v[pl.ds(off, L)]
            d = dst_v[pl.ds(off, L)]
            av = (plsc.load_gather(tbl_v, [r0, s])
                  + plsc.load_gather(tbl_v, [r1, d]))
            av = jnp.maximum(av, 0.2 * av) - shift
            ex = jnp.exp(av)
            h0 = plsc.load_gather(tbl_v, [r2, s])
            h1 = plsc.load_gather(tbl_v, [r3, s])
            plsc.addupdate_scatter(acc_v, [r0, d], ex)
            plsc.addupdate_scatter(acc_v, [r1, d], ex * h0)
            plsc.addupdate_scatter(acc_v, [r2, d], ex * h1)

    with jax.named_scope("sc_selfloop"):
        nbase = pl.multiple_of(wid * NSL, 8)
        nits = (jnp.minimum(NSL, N - nbase) + L - 1) // L

        def loop_step(j, carry):
            sl = pl.ds(nbase + j * L, L)
            av = tbl_v[0, sl] + tbl_v[1, sl]
            av = jnp.maximum(av, 0.2 * av) - shift
            ex = jnp.exp(av)
            plsc.addupdate(acc_v.at[0, sl], ex)
            plsc.addupdate(acc_v.at[1, sl], ex * tbl_v[2, sl])
            plsc.addupdate(acc_v.at[2, sl], ex * tbl_v[3, sl])
            return carry

        lax.fori_loop(0, nits, loop_step, 0)

    with jax.named_scope("sc_combine"):
        plsc.subcore_barrier()
        pltpu.sync_copy(acc_v, sh3.at[i3_v], add=True)
        plsc.subcore_barrier()
    with jax.named_scope("sc_writeback"):
        nb = pl.multiple_of(sid * CHK, 8)
        pltpu.sync_copy(sh3.at[:, pl.ds(nb, CHK)],
                        parts_hbm.at[cid, :, pl.ds(nb, CHK)])


def _epilogue_body(p_hbm, b_ref, out_ref, p_ref):
    pltpu.sync_copy(p_hbm, p_ref)
    p = p_ref[...]
    q = p[:3 * NP2] + p[3 * NP2:]
    den = q[:N]
    n0 = q[NP2:NP2 + N]
    n1 = q[2 * NP2:2 * NP2 + N]
    inv = 1.0 / (den + 1e-16)
    out_ref[...] = jnp.concatenate(
        [(n0 * inv + b_ref[0, 0])[None], (n1 * inv + b_ref[0, 1])[None]],
        axis=0)


@jax.jit
def kernel(x, edge_index, edge_attr, W, att_src, att_dst, bias):
    del edge_attr
    f32 = jnp.float32

    prologue = pl.pallas_call(
        _prologue_body,
        in_specs=[
            pl.BlockSpec((D_IN, N), lambda: (0, 0)),
            pl.BlockSpec((C_OUT, D_IN), lambda: (0, 0)),
            pl.BlockSpec((2, E), lambda: (0, 0)),
            pl.BlockSpec(memory_space=pltpu.SMEM),
            pl.BlockSpec(memory_space=pltpu.SMEM),
        ],
        out_shape=[
            jax.ShapeDtypeStruct((N,), f32),
            jax.ShapeDtypeStruct((N,), f32),
            jax.ShapeDtypeStruct((N,), f32),
            jax.ShapeDtypeStruct((N,), f32),
            jax.ShapeDtypeStruct((L,), f32),
            jax.ShapeDtypeStruct((E,), jnp.int32),
            jax.ShapeDtypeStruct((E,), jnp.int32),
        ],
    )
    asv, adv, h0, h1, sh, src_lin, dst_lin = prologue(
        x.T, W.T, edge_index,
        att_src.reshape(1, C_OUT), att_dst.reshape(1, C_OUT))

    i3 = jnp.asarray(np.arange(3, dtype=np.int32))
    z3 = jnp.asarray(np.zeros((3, NP2), dtype=np.float32))

    mesh = plsc.VectorSubcoreMesh(
        core_axis_name="c", subcore_axis_name="s", num_cores=NC, num_subcores=NS)
    sc = pl.kernel(
        _sc_body,
        out_type=jax.ShapeDtypeStruct((NC, 3, NP2), f32),
        mesh=mesh,
        compiler_params=pltpu.CompilerParams(
            needs_layout_passes=False, use_tc_tiling_on_sc=False),
        scratch_types=[
            pltpu.VMEM((4, N), f32),
            pltpu.VMEM((L,), f32),
            pltpu.VMEM((EPW,), jnp.int32),
            pltpu.VMEM((EPW,), jnp.int32),
            pltpu.VMEM((3, NP2), f32),
            pltpu.VMEM((3,), jnp.int32),
            pltpu.VMEM_SHARED((3, NP2), f32),
            pltpu.VMEM_SHARED((4, N), f32),
            pltpu.SemaphoreType.DMA,
        ],
    )
    parts = sc(asv, adv, h0, h1, sh, src_lin, dst_lin, i3, z3)

    epilogue = pl.pallas_call(
        _epilogue_body,
        in_specs=[
            pl.BlockSpec(memory_space=pl.ANY),
            pl.BlockSpec(memory_space=pltpu.SMEM),
        ],
        scratch_shapes=[pltpu.VMEM((NC * 3 * NP2,), f32)],
        out_shape=jax.ShapeDtypeStruct((2, N), f32),
    )
    out2 = epilogue(parts.reshape(NC * 3 * NP2), bias.reshape(1, C_OUT))
    return out2.T


# speedup vs baseline: 1.7639x; 1.0280x over previous
import jax
import jax.numpy as jnp
import numpy as np
from jax import lax
from jax.experimental import pallas as pl
from jax.experimental.pallas import tpu as pltpu
from jax.experimental.pallas import tpu_sc as plsc

N = 10000
E = 320000
D_IN = 192
C_OUT = 2

NC = 2
NS = 16
L = 16
NW = NC * NS

U = 5
EPW = E // NW
NIT = EPW // (U * L)
NSL = ((N + NW - 1) // NW + L - 1) // L * L
NP2 = 10240
CHK = NP2 // NS
ZIT = NP2 // L


def _prologue_body(xt_ref, wt_ref, ei_ref, as_ref, ad_ref,
                   asv_ref, adv_ref, h0_ref, h1_ref, sh_ref,
                   src_ref, dst_ref):
    ht = lax.dot_general(wt_ref[...], xt_ref[...],
                         dimension_numbers=(((1,), (0,)), ((), ())),
                         preferred_element_type=jnp.float32)
    asv = ht[0:1] * as_ref[0, 0] + ht[1:2] * as_ref[0, 1]
    adv = ht[0:1] * ad_ref[0, 0] + ht[1:2] * ad_ref[0, 1]
    asv_ref[...] = asv[0]
    adv_ref[...] = adv[0]
    h0_ref[...] = ht[0]
    h1_ref[...] = ht[1]
    m = jnp.max(asv) + jnp.max(adv)
    sh = jnp.where(m >= 0.0, m, 0.2 * m)
    sh_ref[...] = jnp.full((L,), sh, jnp.float32)
    src_ref[...] = ei_ref[0]
    dst_ref[...] = ei_ref[1]


def _sc_body(as_hbm, ad_hbm, h0_hbm, h1_hbm, sh_hbm, src_hbm, dst_hbm, i3_hbm,
             z3_hbm, parts_hbm,
             tbl_v, sh_v, src_v, dst_v,
             acc_v, i3_v, sh3, tbl_sh, sem):
    cid = lax.axis_index("c")
    sid = lax.axis_index("s")
    wid = sid * NC + cid
    base = pl.multiple_of(wid * EPW, 8)
    with jax.named_scope("sc_stage"):
        copies = [
            pltpu.async_copy(sh_hbm, sh_v, sem),
            pltpu.async_copy(i3_hbm, i3_v, sem),
            pltpu.async_copy(src_hbm.at[pl.ds(base, EPW)], src_v, sem),
            pltpu.async_copy(dst_hbm.at[pl.ds(base, EPW)], dst_v, sem),
        ]

        @pl.when(sid == 0)
        def _():
            pltpu.sync_copy(z3_hbm, sh3)
        @pl.when(sid == 1)
        def _():
            pltpu.sync_copy(as_hbm, tbl_sh.at[0])
        @pl.when(sid == 2)
        def _():
            pltpu.sync_copy(ad_hbm, tbl_sh.at[1])
        @pl.when(sid == 3)
        def _():
            pltpu.sync_copy(h0_hbm, tbl_sh.at[2])
        @pl.when(sid == 4)
        def _():
            pltpu.sync_copy(h1_hbm, tbl_sh.at[3])

        plsc.subcore_barrier()
        pull = pltpu.async_copy(tbl_sh, tbl_v, sem)

        z = jnp.zeros((L,), jnp.float32)

        @plsc.parallel_loop(0, ZIT, 1, unroll=4)
        def _zero(i):
            off = pl.multiple_of(i * L, 8)
            acc_v[0, pl.ds(off, L)] = z
            acc_v[1, pl.ds(off, L)] = z
            acc_v[2, pl.ds(off, L)] = z

        pull.wait()
        for c in copies:
            c.wait()

    shift = sh_v[...]
    r0 = jnp.zeros((L,), jnp.int32)
    r1 = r0 + 1
    r2 = r0 + 2
    r3 = r0 + 3

    with jax.named_scope("sc_edges"):
        @plsc.parallel_loop(0, EPW // L, 1, unroll=U)
        def _edges(i):
            off = pl.multiple_of(i * L, 8)
            s = src_v[pl.ds(off, L)]
            d = dst_v[pl.ds(off, L)]
            av = (plsc.load_gather(tbl_v, [r0, s])
                  + plsc.load_gather(tbl_v, [r1, d]))
            av = jnp.maximum(av, 0.2 * av) - shift
            ex = jnp.exp(av)
            h0 = plsc.load_gather(tbl_v, [r2, s])
            h1 = plsc.load_gather(tbl_v, [r3, s])
            plsc.addupdate_scatter(acc_v, [r0, d], ex)
            plsc.addupdate_scatter(acc_v, [r1, d], ex * h0)
            plsc.addupdate_scatter(acc_v, [r2, d], ex * h1)

    with jax.named_scope("sc_selfloop"):
        nbase = pl.multiple_of(wid * NSL, 8)
        nits = (jnp.minimum(NSL, N - nbase) + L - 1) // L

        def loop_step(j, carry):
            sl = pl.ds(nbase + j * L, L)
            av = tbl_v[0, sl] + tbl_v[1, sl]
            av = jnp.maximum(av, 0.2 * av) - shift
            ex = jnp.exp(av)
            plsc.addupdate(acc_v.at[0, sl], ex)
            plsc.addupdate(acc_v.at[1, sl], ex * tbl_v[2, sl])
            plsc.addupdate(acc_v.at[2, sl], ex * tbl_v[3, sl])
            return carry

        lax.fori_loop(0, nits, loop_step, 0)

    with jax.named_scope("sc_combine"):
        plsc.subcore_barrier()
        pltpu.sync_copy(acc_v, sh3.at[i3_v], add=True)
        plsc.subcore_barrier()
    with jax.named_scope("sc_writeback"):
        nb = pl.multiple_of(sid * CHK, 8)
        pltpu.sync_copy(sh3.at[:, pl.ds(nb, CHK)],
                        parts_hbm.at[cid, :, pl.ds(nb, CHK)])


def _epilogue_body(p_hbm, b_ref, out_ref, p_ref):
    pltpu.sync_copy(p_hbm, p_ref)
    p = p_ref[...]
    q = p[:3 * NP2] + p[3 * NP2:]
    den = q[:N]
    n0 = q[NP2:NP2 + N]
    n1 = q[2 * NP2:2 * NP2 + N]
    inv = 1.0 / (den + 1e-16)
    out_ref[...] = jnp.concatenate(
        [(n0 * inv + b_ref[0, 0])[None], (n1 * inv + b_ref[0, 1])[None]],
        axis=0)


@jax.jit
def kernel(x, edge_index, edge_attr, W, att_src, att_dst, bias):
    del edge_attr
    f32 = jnp.float32

    prologue = pl.pallas_call(
        _prologue_body,
        in_specs=[
            pl.BlockSpec((D_IN, N), lambda: (0, 0)),
            pl.BlockSpec((C_OUT, D_IN), lambda: (0, 0)),
            pl.BlockSpec((2, E), lambda: (0, 0)),
            pl.BlockSpec(memory_space=pltpu.SMEM),
            pl.BlockSpec(memory_space=pltpu.SMEM),
        ],
        out_shape=[
            jax.ShapeDtypeStruct((N,), f32),
            jax.ShapeDtypeStruct((N,), f32),
            jax.ShapeDtypeStruct((N,), f32),
            jax.ShapeDtypeStruct((N,), f32),
            jax.ShapeDtypeStruct((L,), f32),
            jax.ShapeDtypeStruct((E,), jnp.int32),
            jax.ShapeDtypeStruct((E,), jnp.int32),
        ],
    )
    asv, adv, h0, h1, sh, src_lin, dst_lin = prologue(
        x.T, W.T, edge_index,
        att_src.reshape(1, C_OUT), att_dst.reshape(1, C_OUT))

    i3 = jnp.asarray(np.arange(3, dtype=np.int32))
    z3 = jnp.asarray(np.zeros((3, NP2), dtype=np.float32))

    mesh = plsc.VectorSubcoreMesh(
        core_axis_name="c", subcore_axis_name="s", num_cores=NC, num_subcores=NS)
    sc = pl.kernel(
        _sc_body,
        out_type=jax.ShapeDtypeStruct((NC, 3, NP2), f32),
        mesh=mesh,
        compiler_params=pltpu.CompilerParams(
            needs_layout_passes=False, use_tc_tiling_on_sc=False),
        scratch_types=[
            pltpu.VMEM((4, N), f32),
            pltpu.VMEM((L,), f32),
            pltpu.VMEM((EPW,), jnp.int32),
            pltpu.VMEM((EPW,), jnp.int32),
            pltpu.VMEM((3, NP2), f32),
            pltpu.VMEM((3,), jnp.int32),
            pltpu.VMEM_SHARED((3, NP2), f32),
            pltpu.VMEM_SHARED((4, N), f32),
            pltpu.SemaphoreType.DMA,
        ],
    )
    parts = sc(asv, adv, h0, h1, sh, src_lin, dst_lin, i3, z3)

    epilogue = pl.pallas_call(
        _epilogue_body,
        in_specs=[
            pl.BlockSpec(memory_space=pl.ANY),
            pl.BlockSpec(memory_space=pltpu.SMEM),
        ],
        scratch_shapes=[pltpu.VMEM((NC * 3 * NP2,), f32)],
        out_shape=jax.ShapeDtypeStruct((2, N), f32),
    )
    out2 = epilogue(parts.reshape(NC * 3 * NP2), bias.reshape(1, C_OUT))
    return out2.T
